# Initial kernel scaffold; baseline (speedup 1.0000x reference)
import jax
import jax.numpy as jnp
from jax.experimental import pallas as pl


def kernel(x, adj, kan_W, kan_b, enc_W1, enc_b1, enc_W2, enc_b2, enc_W3, enc_b3, gn_eW1, gn_eb1, gn_eW2, gn_eb2, gn_eW3, gn_eb3, gn_nW1, gn_nb1, gn_nW2, gn_nb2, gn_nW3, gn_nb3, dec_W1, dec_b1, dec_W2, dec_b2, dec_W3, dec_b3):
    raise NotImplementedError("write your pallas kernel here")


# speedup vs baseline: 53.0063x; 53.0063x over previous
import jax
import jax.numpy as jnp
import numpy as np
from jax.experimental import pallas as pl
from jax.experimental.pallas import tpu as pltpu

_N = 512
_HID = 32
_HARM = 5
_NODE_IN = 8
_NBLK = 5
_P = 4
_LN = _P * _HID
_JG = _N // _P
_CI = 8
_ROWS = _CI * _JG


def _gnn_body(xcols, kmul, selo, sels, selc, kan_w, kan_b,
              ew1, eb1, ew2, eb2, ew3, eb3,
              a4, lsel, lselt, pt,
              we1, ws4, wr, we2, we3, be1, be2, be3,
              wn1n, wn1a, wn2, wn3, bn1, bn2, bn3,
              wd1, wd2, wd3, bd1, bd2, bd3,
              out, edge_s, node_s, agg_s, s4_s, r4_s):
    f32 = jnp.float32

    def mm(a, b):
        return jnp.dot(a, b, preferred_element_type=f32)

    y = xcols[...] * kmul[...]
    basis = selo[...] + sels[...] * jnp.sin(y) + selc[...] * jnp.cos(y)
    kb = jnp.sum(kan_b[...], axis=0, keepdims=True)
    node_s[...] = mm(basis, kan_w[...]) + kb

    h = jnp.maximum(jnp.sum(ew1[...], axis=0, keepdims=True) + eb1[...], 0.0)
    h = jnp.maximum(mm(h, ew2[...]) + eb2[...], 0.0)
    e0 = mm(h, ew3[...]) + eb3[...]
    e0b = jnp.concatenate([e0] * _P, axis=1)
    c0 = mm(e0b, we1[0]) + be1[0][None, :]

    for l in range(_NBLK):
        node_u = node_s[...]
        s4_s[...] = mm(node_u, ws4[l])
        u = mm(node_u, wr[l])
        r4_s[...] = jnp.concatenate(
            [mm(lsel[k], u) for k in range(_P)], axis=1)
        agg_s[...] = jnp.zeros((_JG, _LN), f32)

        def chunk(c, carry, l=l):
            r0 = c * _ROWS
            sch = s4_s[pl.ds(c * _CI, _CI), :]
            sterm = jnp.concatenate(
                [jnp.broadcast_to(sch[i:i + 1, :], (_JG, _LN))
                 for i in range(_CI)], axis=0)
            rterm = jnp.concatenate([r4_s[...]] * _CI, axis=0)
            if l == 0:
                h1 = jnp.maximum(sterm + rterm + c0, 0.0)
            else:
                old = edge_s[pl.ds(r0, _ROWS), :]
                h1 = jnp.maximum(mm(old, we1[l]) + sterm + rterm
                                 + be1[l][None, :], 0.0)
            h2 = jnp.maximum(mm(h1, we2[l]) + be2[l][None, :], 0.0)
            d = mm(h2, we3[l]) + be3[l][None, :]
            new = (e0b + d) if l == 0 else (old + d)
            if l < _NBLK - 1:
                edge_s[pl.ds(r0, _ROWS), :] = new
            mn = a4[pl.ds(r0, _ROWS), :].astype(f32) * new
            part = mn[0:_JG, :]
            for i in range(1, _CI):
                part = part + mn[i * _JG:(i + 1) * _JG, :]
            agg_s[...] += part
            return carry

        jax.lax.fori_loop(0, _N // _CI, chunk, 0)

        agg_p = agg_s[...]
        agg_u = mm(lselt[0], mm(agg_p, pt[0]))
        for k in range(1, _P):
            agg_u = agg_u + mm(lselt[k], mm(agg_p, pt[k]))

        h1 = jnp.maximum(mm(node_u, wn1n[l]) + mm(agg_u, wn1a[l])
                         + bn1[l][None, :], 0.0)
        h2 = jnp.maximum(mm(h1, wn2[l]) + bn2[l][None, :], 0.0)
        node_s[...] = node_u + mm(h2, wn3[l]) + bn3[l][None, :]

    node_u = node_s[...]
    h1 = jnp.maximum(mm(node_u, wd1[...]) + bd1[...], 0.0)
    h2 = jnp.maximum(mm(h1, wd2[...]) + bd2[...], 0.0)
    out[...] = mm(h2, wd3[...]) + bd3[...]


def _bd4(w):
    return jnp.kron(jnp.eye(_P, dtype=w.dtype), w)


_NBAS = 2 * _HARM + 1
_COL_I = np.repeat(np.arange(_NODE_IN), _NBAS)
_T = np.tile(np.arange(_NBAS), _NODE_IN)
_KMUL = np.tile(
    np.array([0, 1, 1, 2, 2, 3, 3, 4, 4, 5, 5], np.float32), _NODE_IN)
_SELO = (_T == 0).astype(np.float32)
_SELS = (_T % 2 == 1).astype(np.float32)
_SELC = ((_T > 0) & (_T % 2 == 0)).astype(np.float32)

_LSEL = np.zeros((_P, _JG, _N), np.float32)
for _k in range(_P):
    _LSEL[_k, np.arange(_JG), _P * np.arange(_JG) + _k] = 1.0
_LSELT = np.transpose(_LSEL, (0, 2, 1)).copy()
_PT = np.zeros((_P, _LN, _HID), np.float32)
for _k in range(_P):
    _PT[_k, _HID * _k + np.arange(_HID), np.arange(_HID)] = 1.0


def kernel(x, adj, kan_W, kan_b, enc_W1, enc_b1, enc_W2, enc_b2, enc_W3,
           enc_b3, gn_eW1, gn_eb1, gn_eW2, gn_eb2, gn_eW3, gn_eb3,
           gn_nW1, gn_nb1, gn_nW2, gn_nb2, gn_nW3, gn_nb3,
           dec_W1, dec_b1, dec_W2, dec_b2, dec_W3, dec_b3):
    node_x = x[0, -1]
    a = adj[0]

    a4 = jnp.repeat(a.reshape(_N, _JG, _P).astype(jnp.int8), _HID,
                    axis=2).reshape(_N * _JG, _LN)

    xcols = node_x[:, _COL_I]
    kan_w_flat = kan_W.reshape(_NODE_IN * _NBAS, _HID)

    bd = jax.vmap(_bd4)
    we1 = bd(gn_eW1[:, :_HID, :])
    ws4 = jnp.concatenate([gn_eW1[:, _HID:2 * _HID, :]] * _P, axis=2)
    wr = gn_eW1[:, 2 * _HID:, :]
    we2 = bd(gn_eW2)
    we3 = bd(gn_eW3)
    tile4 = lambda b: jnp.tile(b, (1, _P))
    be1, be2, be3 = tile4(gn_eb1), tile4(gn_eb2), tile4(gn_eb3)

    out = pl.pallas_call(
        _gnn_body,
        out_shape=jax.ShapeDtypeStruct((_N, dec_b3.shape[0]), jnp.float32),
        scratch_shapes=[
            pltpu.VMEM((_N * _JG, _LN), jnp.float32),
            pltpu.VMEM((_N, _HID), jnp.float32),
            pltpu.VMEM((_JG, _LN), jnp.float32),
            pltpu.VMEM((_N, _LN), jnp.float32),
            pltpu.VMEM((_JG, _LN), jnp.float32),
        ],
        compiler_params=pltpu.CompilerParams(
            vmem_limit_bytes=100 * 1024 * 1024),
    )(
        xcols, _KMUL[None, :], _SELO[None, :], _SELS[None, :],
        _SELC[None, :], kan_w_flat, kan_b,
        enc_W1, enc_b1[None, :], enc_W2, enc_b2[None, :], enc_W3,
        enc_b3[None, :],
        a4, _LSEL, _LSELT, _PT,
        we1, ws4, wr, we2, we3, be1, be2, be3,
        gn_nW1[:, :_HID, :], gn_nW1[:, _HID:, :], gn_nW2, gn_nW3,
        gn_nb1, gn_nb2, gn_nb3,
        dec_W1, dec_W2, dec_W3,
        dec_b1[None, :], dec_b2[None, :], dec_b3[None, :],
    )
    return out[None]
